# baseline (speedup 1.0000x reference)
import functools

import jax
import jax.numpy as jnp
from jax import lax
from jax.experimental import pallas as pl
from jax.experimental.pallas import tpu as pltpu
from jax.experimental.pallas import tpu_sc as plsc

_S, _D, _B, _C = 784, 10, 256, 32
_N = _S * _D
_NW = 32
_CH = 5
_UPW = _N // (_NW * _CH)
_NBUF = 3


@functools.partial(
    pl.kernel,
    mesh=plsc.VectorSubcoreMesh(core_axis_name="c", subcore_axis_name="s"),
    out_type=jax.ShapeDtypeStruct((_N, _C, _B), jnp.float32),
    scratch_types=[
        pltpu.VMEM((_UPW, _CH), jnp.int32),
        pltpu.VMEM((_NBUF, _CH, _C, _B), jnp.float32),
        pltpu.SemaphoreType.DMA((_NBUF,)),
        pltpu.SemaphoreType.DMA((_NBUF,)),
    ],
)
def _sc_gather(idx_hbm, x_hbm, out_hbm, idx_v, bufs, gsems, osems):
    wid = lax.axis_index("s") * 2 + lax.axis_index("c")
    base = wid * _UPW
    pltpu.sync_copy(idx_hbm.at[wid], idx_v)

    def g_copy(t):
        b = t % _NBUF
        return pltpu.make_async_copy(
            x_hbm.at[idx_v.at[t]], bufs.at[b], gsems.at[b]
        )

    def o_copy(t):
        b = t % _NBUF
        return pltpu.make_async_copy(
            bufs.at[b], out_hbm.at[pl.ds((base + t) * _CH, _CH)], osems.at[b]
        )

    g_copy(0).start()
    g_copy(1).start()
    for t in range(_UPW):
        if t + 2 < _UPW:
            if t >= 1:
                o_copy(t - 1).wait()
            g_copy(t + 2).start()
        g_copy(t).wait()
        o_copy(t).start()
    o_copy(_UPW - 2).wait()
    o_copy(_UPW - 1).wait()


def kernel(x, permutations):
    S, D, B, C = x.shape
    idx = permutations.T * D + jnp.arange(D, dtype=jnp.int32)[None, :]
    x3 = jnp.transpose(x, (0, 1, 3, 2)).reshape(_N, C, B)
    out = _sc_gather(idx.reshape(_NW, _UPW, _CH), x3)
    return jnp.transpose(out.reshape(S, D, C, B), (0, 1, 3, 2))
